# baseline (speedup 1.0000x reference)
import functools

import numpy as np
import jax
import jax.numpy as jnp
from jax import lax
from jax.experimental import pallas as pl
from jax.experimental.pallas import tpu as pltpu
from jax.experimental.pallas import tpu_sc as plsc

N = 10000
E = 40000
H, FE = 60, 15
HP, FP = 64, 16
HID = HP * FP

NC, NS = 2, 16
NW = NC * NS
EPW = E // NW
GC = 25
NCHUNK = EPW // GC
NPT = 313
CE = 128
CM = 64
EPAD = E + CE
RB = NPT + 23

_MESH = plsc.VectorSubcoreMesh(core_axis_name="c", subcore_axis_name="s")

_OMAP900 = np.arange(H * FE)
_SEL = np.zeros((HID, HP), np.float32)
_SEL[np.arange(H * FE), np.arange(H * FE) // FE] = 1.0


def _wid():
    return lax.axis_index("s") * NC + lax.axis_index("c")


def _mm_body(x_ref, w_ref, b_ref, o_ref, *, relu_in):
    x = x_ref[...]
    if relu_in:
        x = jnp.maximum(x, 0.0)
    o_ref[...] = (
        jnp.dot(x, w_ref[...], preferred_element_type=jnp.float32,
                precision=lax.Precision.HIGHEST) + b_ref[...]
    )


def _mm(x, w, b, relu_in, bm=512, bn=512):
    m, k = x.shape
    n = w.shape[1]
    return pl.pallas_call(
        functools.partial(_mm_body, relu_in=relu_in),
        grid=(pl.cdiv(m, bm), n // bn),
        in_specs=[
            pl.BlockSpec((bm, k), lambda i, j: (i, 0)),
            pl.BlockSpec((k, bn), lambda i, j: (0, j)),
            pl.BlockSpec((1, bn), lambda i, j: (0, j)),
        ],
        out_specs=pl.BlockSpec((bm, bn), lambda i, j: (i, j)),
        out_shape=jax.ShapeDtypeStruct((m, n), jnp.float32),
    )(x, w, b[None])


def _edge_body(rs_ref, rnj_ref, ff_ref, bias_ref, attn_ref, sel_ref, f_ref, e_ref):
    s = rs_ref[...] + rnj_ref[...] + ff_ref[...] + bias_ref[...]
    f = jnp.where(s >= 0, s, 0.01 * s)
    f_ref[...] = f
    e_ref[...] = jnp.dot(
        f * attn_ref[...], sel_ref[...], preferred_element_type=jnp.float32,
        precision=lax.Precision.HIGHEST,
    )


def _edge_stage(rows_src, rows_nj, ffij, bias, attn, sel, bm=512):
    return pl.pallas_call(
        _edge_body,
        grid=(pl.cdiv(EPAD, bm),),
        in_specs=[
            pl.BlockSpec((bm, HID), lambda i: (i, 0)),
            pl.BlockSpec((bm, HID), lambda i: (i, 0)),
            pl.BlockSpec((bm, HID), lambda i: (i, 0)),
            pl.BlockSpec((1, HID), lambda i: (0, 0)),
            pl.BlockSpec((1, HID), lambda i: (0, 0)),
            pl.BlockSpec((HID, HP), lambda i: (0, 0)),
        ],
        out_specs=[
            pl.BlockSpec((bm, HID), lambda i: (i, 0)),
            pl.BlockSpec((bm, HP), lambda i: (i, 0)),
        ],
        out_shape=[
            jax.ShapeDtypeStruct((E, HID), jnp.float32),
            jax.ShapeDtypeStruct((EPAD, HP), jnp.float32),
        ],
    )(rows_src, rows_nj, ffij, bias[None], attn[None], sel)


def _msg_body(a_ref, rh_ref, selt_ref, m_ref):
    aexp = jnp.dot(a_ref[...], selt_ref[...], preferred_element_type=jnp.float32,
                   precision=lax.Precision.HIGHEST)
    m_ref[...] = aexp * rh_ref[...]


def _msg_stage(a, rows_src, selt, bm=512):
    return pl.pallas_call(
        _msg_body,
        grid=(pl.cdiv(EPAD, bm),),
        in_specs=[
            pl.BlockSpec((bm, HP), lambda i: (i, 0)),
            pl.BlockSpec((bm, HID), lambda i: (i, 1)),
            pl.BlockSpec((HP, HID), lambda i: (0, 0)),
        ],
        out_specs=pl.BlockSpec((bm, HID), lambda i: (i, 0)),
        out_shape=jax.ShapeDtypeStruct((EPAD, HID), jnp.float32),
    )(a, rows_src, selt)


def _gather(table, idx3, width):
    nrows = table.shape[0]

    @functools.partial(
        pl.kernel,
        out_type=jax.ShapeDtypeStruct((E, width), jnp.float32),
        mesh=_MESH,
        compiler_params=pltpu.CompilerParams(use_tc_tiling_on_sc=False),
        scratch_types=[
            pltpu.VMEM((GC,), jnp.int32),
            pltpu.VMEM((GC, width), jnp.float32),
            pltpu.SemaphoreType.DMA,
        ],
    )
    def gk(tab_hbm, idx_hbm, out_hbm, idxbuf, rbuf, sem):
        w = _wid()

        def chunk(c, _):
            pltpu.sync_copy(idx_hbm.at[w, c], idxbuf)
            pltpu.async_copy(tab_hbm.at[idxbuf], rbuf, sem).wait()
            pltpu.sync_copy(rbuf, out_hbm.at[pl.ds(w * EPW + c * GC, GC)])
            return 0

        lax.fori_loop(0, NCHUNK, chunk, 0)

    return gk(table, idx3)


@functools.partial(
    pl.kernel,
    out_type=jax.ShapeDtypeStruct((EPAD, HP), jnp.float32),
    mesh=_MESH,
    compiler_params=pltpu.CompilerParams(use_tc_tiling_on_sc=False),
    scratch_types=[
        pltpu.VMEM((16,), jnp.int32),
        pltpu.VMEM((CE, 16), jnp.int32),
        pltpu.VMEM((CE, HP), jnp.float32),
        pltpu.VMEM((CE, HP), jnp.float32),
        pltpu.VMEM((NPT + 16, HP), jnp.float32),
        pltpu.VMEM((NPT + 16, HP), jnp.float32),
        pltpu.VMEM((2 * HP,), jnp.float32),
        pltpu.SemaphoreType.DMA,
    ],
)
def _softmax_kernel(e_hbm, dst_hbm, tb_hbm, a_hbm, tbuf, dbuf, ebuf, abuf,
                    smax, rden, run, sem):
    w = _wid()
    pltpu.sync_copy(tb_hbm.at[w], tbuf)
    tv = tbuf[pl.ds(0, 16)]
    estart = tv[0]
    eend = tv[1]
    nstart = w * NPT
    nch = lax.div(eend - estart + CE - 1, CE)

    neg = jnp.full((16,), -1e30, jnp.float32)
    zero = jnp.zeros((16,), jnp.float32)

    def reset_run():
        for k in range(4):
            run[pl.ds(k * 16, 16)] = neg
            run[pl.ds(HP + k * 16, 16)] = zero

    reset_run()

    def finalize(p):
        ln = p - nstart
        for k in range(4):
            smax[ln, pl.ds(k * 16, 16)] = run[pl.ds(k * 16, 16)]
            rden[ln, pl.ds(k * 16, 16)] = 1.0 / jnp.maximum(
                run[pl.ds(HP + k * 16, 16)], 1e-9)
        reset_run()

    def edge_a(i, p):
        q = dbuf[i, pl.ds(0, 16)][0]

        @pl.when(jnp.logical_and(q != p, p >= nstart))
        def _():
            finalize(p)

        for k in range(4):
            x = ebuf[i, pl.ds(k * 16, 16)]
            m = run[pl.ds(k * 16, 16)]
            mn = jnp.maximum(m, x)
            run[pl.ds(HP + k * 16, 16)] = (
                run[pl.ds(HP + k * 16, 16)] * jnp.exp(m - mn) + jnp.exp(x - mn))
            run[pl.ds(k * 16, 16)] = mn
        return q

    def chunk_a(c, p):
        base = estart + c * CE
        pltpu.sync_copy(e_hbm.at[pl.ds(base, CE)], ebuf)
        pltpu.sync_copy(dst_hbm.at[pl.ds(base, CE)], dbuf)
        cnt = jnp.minimum(CE, eend - base)
        return lax.fori_loop(0, cnt, edge_a, p)

    p = lax.fori_loop(0, nch, chunk_a, jnp.int32(nstart - 1))

    @pl.when(p >= nstart)
    def _():
        finalize(p)

    def edge_b(i, _):
        ln = dbuf[i, pl.ds(0, 16)][0] - nstart
        for k in range(4):
            x = ebuf[i, pl.ds(k * 16, 16)]
            mm = smax[ln, pl.ds(k * 16, 16)]
            rd = rden[ln, pl.ds(k * 16, 16)]
            abuf[i, pl.ds(k * 16, 16)] = jnp.exp(x - mm) * rd
        return 0

    def chunk_b(c, _):
        base = estart + c * CE
        pltpu.sync_copy(e_hbm.at[pl.ds(base, CE)], ebuf)
        pltpu.sync_copy(dst_hbm.at[pl.ds(base, CE)], dbuf)
        cnt = jnp.minimum(CE, eend - base)
        lax.fori_loop(0, cnt, edge_b, 0)

        @pl.when(cnt == CE)
        def _():
            pltpu.sync_copy(abuf, a_hbm.at[pl.ds(base, CE)])

        @pl.when(cnt < CE)
        def _():
            def wr(r, _):
                pltpu.sync_copy(abuf.at[r], a_hbm.at[base + r])
                return 0

            lax.fori_loop(0, cnt, wr, 0)

        return 0

    lax.fori_loop(0, nch, chunk_b, 0)


@functools.partial(
    pl.kernel,
    out_type=jax.ShapeDtypeStruct((N, HID), jnp.float32),
    mesh=_MESH,
    compiler_params=pltpu.CompilerParams(use_tc_tiling_on_sc=False),
    scratch_types=[
        pltpu.VMEM((16,), jnp.int32),
        pltpu.VMEM((CM, 16), jnp.int32),
        pltpu.VMEM((CM, HID), jnp.float32),
        pltpu.VMEM((HID,), jnp.float32),
        pltpu.VMEM((HID,), jnp.float32),
        pltpu.SemaphoreType.DMA,
    ],
)
def _segsum_kernel(m_hbm, dst_hbm, tb_hbm, out_hbm, tbuf, dbuf, mbuf, acc,
                   zbuf, sem):
    w = _wid()
    pltpu.sync_copy(tb_hbm.at[w], tbuf)
    tv = tbuf[pl.ds(0, 16)]
    estart = tv[0]
    eend = tv[1]
    nstart = w * NPT
    nend = jnp.minimum(nstart + NPT, N)
    nch = lax.div(eend - estart + CM - 1, CM)
    zero = jnp.zeros((16,), jnp.float32)

    for k in range(64):
        acc[pl.ds(k * 16, 16)] = zero
        zbuf[pl.ds(k * 16, 16)] = zero

    def zero_row(n, _):
        pltpu.sync_copy(zbuf, out_hbm.at[n])
        return 0

    def edge(i, p):
        q = dbuf[i, pl.ds(0, 16)][0]

        @pl.when(q != p)
        def _():
            @pl.when(p >= nstart)
            def _():
                pltpu.sync_copy(acc, out_hbm.at[p])
                for k in range(64):
                    acc[pl.ds(k * 16, 16)] = zero

            lax.fori_loop(jnp.maximum(p + 1, nstart), q, zero_row, 0)

        for k in range(64):
            sl = pl.ds(k * 16, 16)
            acc[sl] = acc[sl] + mbuf[i, sl]
        return q

    def chunk(c, p):
        base = estart + c * CM
        pltpu.sync_copy(m_hbm.at[pl.ds(base, CM)], mbuf)
        pltpu.sync_copy(dst_hbm.at[pl.ds(base, CM)], dbuf)
        cnt = jnp.minimum(CM, eend - base)
        return lax.fori_loop(0, cnt, edge, p)

    p = lax.fori_loop(0, nch, chunk, jnp.int32(nstart - 1))

    @pl.when(p >= nstart)
    def _():
        pltpu.sync_copy(acc, out_hbm.at[p])

    lax.fori_loop(jnp.maximum(p + 1, nstart), nend, zero_row, 0)


def _expand_w(w_real, in_map, out_map, kin):
    wp = jnp.zeros((kin, HID), jnp.float32)
    return wp.at[jnp.ix_(in_map, out_map)].set(w_real.T)


def _expand_v(v_real, out_map):
    return jnp.zeros((HID,), jnp.float32).at[out_map].set(v_real)


def _prep_layer(p, lidx):
    if lidx == 0:
        in_map = jnp.arange(3)
        kin = 8
    else:
        in_map = jnp.asarray(_OMAP900)
        kin = HID
    in_map_e = jnp.arange(1) if lidx == 0 else jnp.asarray(_OMAP900)
    if lidx == 8:
        out_map = jnp.arange(1)
    else:
        out_map = jnp.asarray(_OMAP900)
    w_ni = _expand_w(p["W_ni"], in_map, out_map, kin)
    w_node = _expand_w(p["W_node"], in_map, out_map, kin)
    w_nj = _expand_w(p["W_nj"], in_map, out_map, kin)
    w_fij = _expand_w(p["W_fij"], in_map_e, out_map, kin)
    wcat = jnp.concatenate([w_ni, w_node], axis=1)
    bcat = jnp.concatenate(
        [jnp.zeros((HID,), jnp.float32), _expand_v(p["b_node"], out_map)])
    bias = _expand_v(p["bias"], out_map)
    attn = _expand_v(p["attn"].reshape(-1), out_map)
    return wcat, bcat, w_nj, w_fij, bias, attn


def kernel(node_f, edge_f, edge_index, params):
    src = edge_index[0]
    dst = edge_index[1]
    perm = jnp.argsort(dst)
    dst_s = dst[perm]
    src_s = src[perm]
    rowptr = jnp.searchsorted(
        dst_s, jnp.arange(N + 1, dtype=jnp.int32), side="left"
    ).astype(jnp.int32)
    nstarts = jnp.minimum(jnp.arange(NW, dtype=jnp.int32) * NPT, N)
    nends = jnp.minimum(nstarts + NPT, N)
    tb = jnp.zeros((NW, 16), jnp.int32)
    tb = tb.at[:, 0].set(rowptr[nstarts]).at[:, 1].set(rowptr[nends])
    dst16 = jnp.zeros((EPAD, 16), jnp.int32).at[:E, :].set(
        jnp.broadcast_to(dst_s[:, None], (E, 16)))
    src3 = src_s.reshape(NW, NCHUNK, GC)
    dst3 = dst_s.reshape(NW, NCHUNK, GC)
    inv = jnp.zeros((E,), jnp.int32).at[perm].set(
        jnp.arange(E, dtype=jnp.int32)).reshape(NW, NCHUNK, GC)

    sel = jnp.asarray(_SEL)
    selt = sel.T

    nf = jnp.pad(node_f, ((0, 0), (0, 5)))
    ef = jnp.pad(edge_f[perm], ((0, 0), (0, 7)))

    h_out = None
    f_out = None
    for lidx, p in enumerate(params):
        wcat, bcat, w_nj, w_fij, bias, attn = _prep_layer(p, lidx)
        relu_in = lidx > 0
        t12 = _mm(nf, wcat, bcat, relu_in)
        tnj = _mm(nf, w_nj, jnp.zeros((HID,), jnp.float32), relu_in)
        ffij = _mm(ef, w_fij, jnp.zeros((HID,), jnp.float32), relu_in)
        rows_src = _gather(t12, src3, 2048)
        rows_nj = _gather(tnj, dst3, HID)
        f_out, e = _edge_stage(rows_src, rows_nj, ffij, bias, attn, sel)
        a = _softmax_kernel(e, dst16, tb)
        m = _msg_stage(a, rows_src, selt)
        h_out = _segsum_kernel(m, dst16, tb)
        nf = h_out
        ef = f_out

    f_uns = _gather(f_out, inv, HID)
    return (
        h_out[:, :1].reshape(N, 1, 1),
        f_uns[:, :1].reshape(E, 1, 1),
    )
